# baseline (speedup 1.0000x reference)
import functools

import jax
import jax.numpy as jnp
from jax import lax
from jax.experimental import pallas as pl
from jax.experimental.pallas import tpu as pltpu
from jax.experimental.pallas import tpu_sc as plsc

VOCAB = 100000
EMB = 32
CTX = 20
B = 1024
HID = 128
DIN = CTX * EMB

NC, NS = 2, 16
NW = NC * NS
NIDX = B * CTX
IDX_PER_W = NIDX // NW

VT = 1024
NV = (VOCAB + VT - 1) // VT


def _gather_body(table_hbm, idx_hbm, out_hbm, idx_v, rows_v, sem):
    wid = lax.axis_index("s") * NC + lax.axis_index("c")
    base = wid * IDX_PER_W
    pltpu.sync_copy(idx_hbm.at[pl.ds(base, IDX_PER_W)], idx_v)
    pltpu.async_copy(table_hbm.at[idx_v], rows_v, sem).wait()
    pltpu.sync_copy(rows_v, out_hbm.at[pl.ds(base, IDX_PER_W)])


def _make_gather():
    return functools.partial(
        pl.kernel,
        mesh=plsc.VectorSubcoreMesh(core_axis_name="c", subcore_axis_name="s"),
        out_type=jax.ShapeDtypeStruct((NIDX, EMB), jnp.float32),
        scratch_types=[
            pltpu.VMEM((IDX_PER_W,), jnp.int32),
            pltpu.VMEM((IDX_PER_W, EMB), jnp.float32),
            pltpu.SemaphoreType.DMA,
        ],
        compiler_params=pltpu.CompilerParams(use_tc_tiling_on_sc=False),
    )(_gather_body)


def _mlp_lsm_body(emb_ref, w1_ref, b1_ref, w2_ref, b2_ref, out_ref,
                  h_ref, m_ref, s_ref):
    p = pl.program_id(0)
    j = pl.program_id(1)

    @pl.when((p == 0) & (j == 0))
    def _init():
        h = lax.dot_general(emb_ref[...], w1_ref[...], (((1,), (1,)), ((), ())),
                            preferred_element_type=jnp.float32)
        h_ref[...] = jnp.maximum(h + b1_ref[...], 0.0).astype(jnp.bfloat16)
        m_ref[...] = jnp.full((B, 1), -1e30, jnp.float32)
        s_ref[...] = jnp.zeros((B, 1), jnp.float32)

    logits = lax.dot_general(h_ref[...], w2_ref[...].astype(jnp.bfloat16),
                             (((1,), (1,)), ((), ())),
                             preferred_element_type=jnp.float32)
    logits = logits + b2_ref[...]
    cols = j * VT + lax.broadcasted_iota(jnp.int32, (1, VT), 1)
    logits = jnp.where(cols < VOCAB, logits, -1e30)

    @pl.when(p == 0)
    def _pass0():
        tmax = jnp.max(logits, axis=1, keepdims=True)
        mnew = jnp.maximum(m_ref[...], tmax)
        s_ref[...] = (s_ref[...] * jnp.exp(m_ref[...] - mnew)
                      + jnp.sum(jnp.exp(logits - mnew), axis=1, keepdims=True))
        m_ref[...] = mnew

    @pl.when(p == 1)
    def _pass1():
        out_ref[...] = logits - (m_ref[...] + jnp.log(s_ref[...]))


def kernel(context_words, emb_table, W1, b1, W2, b2):
    idx = context_words.reshape(NIDX)
    rows = _make_gather()(emb_table, idx)
    emb = rows.reshape(B, DIN)

    out = pl.pallas_call(
        _mlp_lsm_body,
        grid=(2, NV),
        in_specs=[
            pl.BlockSpec((B, DIN), lambda p, j: (0, 0)),
            pl.BlockSpec((HID, DIN), lambda p, j: (0, 0)),
            pl.BlockSpec((1, HID), lambda p, j: (0, 0)),
            pl.BlockSpec((VT, HID), lambda p, j: (j, 0)),
            pl.BlockSpec((1, VT), lambda p, j: (0, j)),
        ],
        out_specs=pl.BlockSpec((B, VT),
                               lambda p, j: (0, jnp.where(p == 1, j, 0))),
        out_shape=jax.ShapeDtypeStruct((B, VOCAB), jnp.float32),
        scratch_shapes=[
            pltpu.VMEM((B, HID), jnp.bfloat16),
            pltpu.VMEM((B, 1), jnp.float32),
            pltpu.VMEM((B, 1), jnp.float32),
        ],
        compiler_params=pltpu.CompilerParams(
            dimension_semantics=("arbitrary", "arbitrary")),
    )(emb, W1, b1.reshape(1, HID), W2, b2.reshape(1, VOCAB))
    return out
